# Initial kernel scaffold; baseline (speedup 1.0000x reference)
import jax
import jax.numpy as jnp
from jax.experimental import pallas as pl


def kernel(x, edge_index, edge_attr, params):
    raise NotImplementedError("write your pallas kernel here")


# speedup vs baseline: 2.2135x; 2.2135x over previous
import functools

import jax
import jax.numpy as jnp
from jax import lax
from jax.experimental import pallas as pl
from jax.experimental.pallas import tpu as pltpu
from jax.experimental.pallas import tpu_sc as plsc

_SC_PARAMS = pltpu.CompilerParams(use_tc_tiling_on_sc=False)

N = 50000
E = 800000
EMB = 32
RADIUS = 0.075

NC = 2
NS = 16
NW = NC * NS
GCH = 128
K2 = 8

EPAD = 819200
M2 = 2 * EPAD
NPAD = 50176
BE = 2048
BN = 2048
NEB = EPAD // BE
NNB = (N + BN - 1) // BN


def _gather_body(rounds, table_hbm, idx_hbm, out_hbm, idx_v, rows_v, sg, so0, so1):
    wid = lax.axis_index("c") * NS + lax.axis_index("s")
    base = wid * (rounds * K2)
    pltpu.sync_copy(idx_hbm.at[pl.ds(base, rounds * K2)], idx_v)
    so = (so0, so1)

    @pl.loop(0, rounds, step=2)
    def _(r0):
        for t in range(2):
            r = r0 + t

            @pl.when(r >= 2)
            def _():
                pltpu.make_async_copy(
                    rows_v.at[t], out_hbm.at[pl.ds(0, K2 * GCH)], so[t]).wait()

            handles = [
                pltpu.async_copy(
                    table_hbm.at[idx_v.at[r * K2 + k]],
                    rows_v.at[t, pl.ds(k * GCH, GCH)], sg)
                for k in range(K2)
            ]
            for h in handles:
                h.wait()
            pltpu.async_copy(
                rows_v.at[t],
                out_hbm.at[pl.ds((base + r * K2) * GCH, K2 * GCH)], so[t])

    for t in range(2):
        pltpu.make_async_copy(
            rows_v.at[t], out_hbm.at[pl.ds(0, K2 * GCH)], so[t]).wait()


def _make_gather(n_rows, d, m):
    rounds = m // (NW * GCH * K2)
    mesh = plsc.VectorSubcoreMesh(
        core_axis_name="c", subcore_axis_name="s", num_cores=NC, num_subcores=NS)
    return pl.kernel(
        functools.partial(_gather_body, rounds),
        out_type=jax.ShapeDtypeStruct((m, d), jnp.float32),
        mesh=mesh,
        compiler_params=_SC_PARAMS,
        scratch_types=[
            pltpu.VMEM((m // (NW * GCH), GCH), jnp.int32),
            pltpu.VMEM((2, K2 * GCH, d), jnp.float32),
            pltpu.SemaphoreType.DMA,
            pltpu.SemaphoreType.DMA,
            pltpu.SemaphoreType.DMA,
        ],
    )


_TPW = NPAD // NS
_ZCH = _TPW // 7
_SG = 4


def _scatter_body(msg_hbm, idx_hbm, zeros_hbm, out_hbm, idx_v, rows_v, acc):
    c = lax.axis_index("c")
    s = lax.axis_index("s")
    pltpu.sync_copy(zeros_hbm, rows_v.at[pl.ds(0, _ZCH)])
    for q in range(7):
        pltpu.sync_copy(rows_v.at[pl.ds(0, _ZCH)],
                        acc.at[pl.ds(s * _TPW + q * _ZCH, _ZCH)])
    plsc.subcore_barrier()

    chunks = EPAD // (NW * GCH)
    base = (c * NS + s) * chunks

    @pl.loop(0, chunks // _SG)
    def _(g):
        pltpu.sync_copy(idx_hbm.at[pl.ds(base + g * _SG, _SG)], idx_v)
        pltpu.sync_copy(
            msg_hbm.at[pl.ds((base + g * _SG) * GCH, _SG * GCH)], rows_v)
        for k in range(_SG):
            pltpu.sync_copy(
                rows_v.at[pl.ds(k * GCH, GCH)],
                acc.at[idx_v.at[k]], add=True)

    plsc.subcore_barrier()
    for q in range(7):
        row0 = s * _TPW + q * _ZCH
        pltpu.sync_copy(acc.at[pl.ds(row0, _ZCH)], rows_v.at[pl.ds(0, _ZCH)])
        pltpu.sync_copy(rows_v.at[pl.ds(0, _ZCH)],
                        out_hbm.at[c].at[pl.ds(row0, _ZCH)])


def _make_scatter():
    mesh = plsc.VectorSubcoreMesh(
        core_axis_name="c", subcore_axis_name="s", num_cores=NC, num_subcores=NS)
    return pl.kernel(
        _scatter_body,
        out_type=jax.ShapeDtypeStruct((NC, NPAD, EMB), jnp.float32),
        mesh=mesh,
        compiler_params=_SC_PARAMS,
        scratch_types=[
            pltpu.VMEM((_SG, GCH), jnp.int32),
            pltpu.VMEM((_SG * GCH, EMB), jnp.float32),
            pltpu.VMEM_SHARED((NPAD, EMB), jnp.float32),
        ],
    )


def _dot(a, w):
    return jnp.dot(a, w, preferred_element_type=jnp.float32)


def _mlp3(t, w1, b1, w2, b2, w3, b3):
    t = jnp.maximum(_dot(t, w1) + b1, 0.0)
    t = jnp.maximum(_dot(t, w2) + b2, 0.0)
    return _dot(t, w3) + b3


def _node_embed_body(x_ref, w1, b1, w2, b2, w3, b3, o_ref):
    x = x_ref[...]
    nf = jnp.concatenate(
        [x[:, 0:4]] + [x[:, 4:5]] * 4, axis=-1)
    o_ref[...] = _mlp3(nf, w1[...], b1[...], w2[...], b2[...], w3[...], b3[...])


def _edge_embed_body(xi_ref, xj_ref, w1, b1, w2, b2, w3, b3, o_ref):
    xi = xi_ref[...]
    xj = xj_ref[...]
    dpos = xi[:, 0:2] - xj[:, 0:2]
    r = jnp.sqrt(jnp.sum(dpos * dpos, axis=1, keepdims=True)) / RADIUS
    feat = jnp.concatenate(
        [dpos / RADIUS, r, xi[:, 2:4], xj[:, 2:4]] + [xi[:, 4:5]] * 4 +
        [jnp.zeros_like(xi[:, 0:5])], axis=-1)
    o_ref[...] = _mlp3(feat, w1[...], b1[...], w2[...], b2[...], w3[...], b3[...])


def _edge_layer_body(e_ref, hd_ref, hs_ref, w1, b1, w2, b2, w3, b3,
                     msg_ref, enew_ref):
    i = pl.program_id(0)
    eb = e_ref[...]
    t = jnp.concatenate([eb, hd_ref[...], hs_ref[...]], axis=-1)
    msg = _mlp3(t, w1[...], b1[...], w2[...], b2[...], w3[...], b3[...])
    row = i * BE + lax.broadcasted_iota(jnp.int32, (BE, 1), 0)
    msg = jnp.where(row < E, msg, 0.0)
    msg_ref[...] = msg
    enew_ref[...] = eb + msg


def _node_layer_body(h_ref, p0_ref, p1_ref, w1, b1, w2, b2, w3, b3, o_ref):
    h = h_ref[...]
    aggr = p0_ref[0] + p1_ref[0]
    t = jnp.concatenate([h, aggr], axis=-1)
    o_ref[...] = h + _mlp3(t, w1[...], b1[...], w2[...], b2[...], w3[...], b3[...])


def _node_out_body(h_ref, w1, b1, w2, b2, w3, b3, o_ref):
    o_ref[...] = _mlp3(h_ref[...], w1[...], b1[...], w2[...], b2[...],
                       w3[...], b3[...])


def _wspecs(ws):
    return [pl.BlockSpec(w.shape, lambda i, nd=w.ndim: (0,) * nd) for w in ws]


def _edge_call(body, ws, n_out):
    eb = pl.BlockSpec((BE, EMB), lambda i: (i, 0))
    out_shape = [jax.ShapeDtypeStruct((EPAD, EMB), jnp.float32)] * n_out
    return pl.pallas_call(
        body,
        grid=(NEB,),
        in_specs=[eb,
                  pl.BlockSpec((BE, EMB), lambda i: (i, 0)),
                  pl.BlockSpec((BE, EMB), lambda i: (i + NEB, 0))] + _wspecs(ws),
        out_specs=[eb] * n_out,
        out_shape=out_shape,
    )


def kernel(x, edge_index, edge_attr, params):
    del edge_attr
    src = edge_index[0]
    dst = edge_index[1]
    zeros_i = jnp.zeros((EPAD - E,), jnp.int32)
    dst_p = jnp.concatenate([dst, zeros_i])
    src_p = jnp.concatenate([src, zeros_i])
    gidx = jnp.concatenate([dst_p, src_p]).reshape(M2 // GCH, GCH)
    dst2d = dst_p.reshape(EPAD // GCH, GCH)
    xpad = jnp.pad(x, ((0, 0), (0, 11)))
    zrows = jnp.zeros((_ZCH, EMB), jnp.float32)

    p_ne = [t for wb in params['embedding_node'] for t in wb]
    we1, be1 = params['embedding_edges'][0]
    we1 = jnp.pad(we1, ((0, 5), (0, 0)))
    p_ee = [we1, be1] + [t for wb in params['embedding_edges'][1:] for t in wb]
    p_out = [t for wb in params['node_out'] for t in wb]

    gx = _make_gather(N, 16, M2)(xpad, gidx)

    h = pl.pallas_call(
        _node_embed_body,
        grid=(NNB,),
        in_specs=[pl.BlockSpec((BN, 5), lambda i: (i, 0))] + _wspecs(p_ne),
        out_specs=pl.BlockSpec((BN, EMB), lambda i: (i, 0)),
        out_shape=jax.ShapeDtypeStruct((N, EMB), jnp.float32),
    )(x, *p_ne)

    e = pl.pallas_call(
        _edge_embed_body,
        grid=(NEB,),
        in_specs=[pl.BlockSpec((BE, 16), lambda i: (i, 0)),
                  pl.BlockSpec((BE, 16), lambda i: (i + NEB, 0))] + _wspecs(p_ee),
        out_specs=pl.BlockSpec((BE, EMB), lambda i: (i, 0)),
        out_shape=jax.ShapeDtypeStruct((EPAD, EMB), jnp.float32),
    )(gx, gx, *p_ee)

    gather_h = _make_gather(N, EMB, M2)
    scatter = _make_scatter()

    for lp in params['layers']:
        p_edge = [t for wb in lp['lin_edge'] for t in wb]
        p_node = [t for wb in lp['lin_node'] for t in wb]

        gh = gather_h(h, gidx)
        msg, e = _edge_call(_edge_layer_body, p_edge, 2)(e, gh, gh, *p_edge)
        partials = scatter(msg, dst2d, zrows)
        h = pl.pallas_call(
            _node_layer_body,
            grid=(NNB,),
            in_specs=[pl.BlockSpec((BN, EMB), lambda i: (i, 0)),
                      pl.BlockSpec((1, BN, EMB), lambda i: (0, i, 0)),
                      pl.BlockSpec((1, BN, EMB), lambda i: (1, i, 0))] +
                     _wspecs(p_node),
            out_specs=pl.BlockSpec((BN, EMB), lambda i: (i, 0)),
            out_shape=jax.ShapeDtypeStruct((N, EMB), jnp.float32),
        )(h, partials, partials, *p_node)

    pred = pl.pallas_call(
        _node_out_body,
        grid=(NNB,),
        in_specs=[pl.BlockSpec((BN, EMB), lambda i: (i, 0))] + _wspecs(p_out),
        out_specs=pl.BlockSpec((BN, 2), lambda i: (i, 0)),
        out_shape=jax.ShapeDtypeStruct((N, 2), jnp.float32),
    )(h, *p_out)
    return pred


# speedup vs baseline: 2.2781x; 1.0292x over previous
import functools

import jax
import jax.numpy as jnp
from jax import lax
from jax.experimental import pallas as pl
from jax.experimental.pallas import tpu as pltpu
from jax.experimental.pallas import tpu_sc as plsc

_SC_PARAMS = pltpu.CompilerParams(use_tc_tiling_on_sc=False)

N = 50000
E = 800000
EMB = 32
RADIUS = 0.075

NC = 2
NS = 16
NW = NC * NS
GCH = 128
K2 = 8

EPAD = 819200
M2 = 2 * EPAD
NPAD = 50176
BE = 2048
BN = 2048
NEB = EPAD // BE
NNB = (N + BN - 1) // BN


def _gather_body(rounds, table_hbm, idx_hbm, out_hbm, idx_v, rows_v,
                 sg0, sg1, so0, so1):
    wid = lax.axis_index("c") * NS + lax.axis_index("s")
    base = wid * (rounds * K2)
    pltpu.sync_copy(idx_hbm.at[pl.ds(base, rounds * K2)], idx_v)
    sg = (sg0, sg1)
    so = (so0, so1)

    def fire(r, t):
        for k in range(K2):
            pltpu.async_copy(
                table_hbm.at[idx_v.at[r * K2 + k]],
                rows_v.at[t, pl.ds(k * GCH, GCH)], sg[t])

    def drain_gathers(t):
        for _ in range(K2):
            pltpu.make_async_copy(
                table_hbm.at[idx_v.at[0]], rows_v.at[t, pl.ds(0, GCH)],
                sg[t]).wait()

    fire(0, 0)

    @pl.loop(0, rounds, step=2)
    def _(r0):
        for t in range(2):
            r = r0 + t
            nxt = 1 - t

            @pl.when(r + 1 < rounds)
            def _():
                @pl.when(r >= 1)
                def _():
                    pltpu.make_async_copy(
                        rows_v.at[nxt], out_hbm.at[pl.ds(0, K2 * GCH)],
                        so[nxt]).wait()

                fire(r + 1, nxt)

            drain_gathers(t)
            pltpu.async_copy(
                rows_v.at[t],
                out_hbm.at[pl.ds((base + r * K2) * GCH, K2 * GCH)], so[t])

    for t in range(2):
        pltpu.make_async_copy(
            rows_v.at[t], out_hbm.at[pl.ds(0, K2 * GCH)], so[t]).wait()


def _make_gather(n_rows, d, m):
    rounds = m // (NW * GCH * K2)
    mesh = plsc.VectorSubcoreMesh(
        core_axis_name="c", subcore_axis_name="s", num_cores=NC, num_subcores=NS)
    return pl.kernel(
        functools.partial(_gather_body, rounds),
        out_type=jax.ShapeDtypeStruct((m, d), jnp.float32),
        mesh=mesh,
        compiler_params=_SC_PARAMS,
        scratch_types=[
            pltpu.VMEM((m // (NW * GCH), GCH), jnp.int32),
            pltpu.VMEM((2, K2 * GCH, d), jnp.float32),
            pltpu.SemaphoreType.DMA,
            pltpu.SemaphoreType.DMA,
            pltpu.SemaphoreType.DMA,
            pltpu.SemaphoreType.DMA,
        ],
    )


_TPW = NPAD // NS
_ZCH = _TPW // 7
_SG = 4


def _scatter_body(msg_hbm, idx_hbm, zeros_hbm, out_hbm, idx_v, rows_v, acc):
    c = lax.axis_index("c")
    s = lax.axis_index("s")
    pltpu.sync_copy(zeros_hbm, rows_v.at[pl.ds(0, _ZCH)])
    for q in range(7):
        pltpu.sync_copy(rows_v.at[pl.ds(0, _ZCH)],
                        acc.at[pl.ds(s * _TPW + q * _ZCH, _ZCH)])
    plsc.subcore_barrier()

    chunks = EPAD // (NW * GCH)
    base = (c * NS + s) * chunks

    @pl.loop(0, chunks // _SG)
    def _(g):
        pltpu.sync_copy(idx_hbm.at[pl.ds(base + g * _SG, _SG)], idx_v)
        pltpu.sync_copy(
            msg_hbm.at[pl.ds((base + g * _SG) * GCH, _SG * GCH)], rows_v)
        for k in range(_SG):
            pltpu.sync_copy(
                rows_v.at[pl.ds(k * GCH, GCH)],
                acc.at[idx_v.at[k]], add=True)

    plsc.subcore_barrier()
    for q in range(7):
        row0 = s * _TPW + q * _ZCH
        pltpu.sync_copy(acc.at[pl.ds(row0, _ZCH)], rows_v.at[pl.ds(0, _ZCH)])
        pltpu.sync_copy(rows_v.at[pl.ds(0, _ZCH)],
                        out_hbm.at[c].at[pl.ds(row0, _ZCH)])


def _make_scatter():
    mesh = plsc.VectorSubcoreMesh(
        core_axis_name="c", subcore_axis_name="s", num_cores=NC, num_subcores=NS)
    return pl.kernel(
        _scatter_body,
        out_type=jax.ShapeDtypeStruct((NC, NPAD, EMB), jnp.float32),
        mesh=mesh,
        compiler_params=_SC_PARAMS,
        scratch_types=[
            pltpu.VMEM((_SG, GCH), jnp.int32),
            pltpu.VMEM((_SG * GCH, EMB), jnp.float32),
            pltpu.VMEM_SHARED((NPAD, EMB), jnp.float32),
        ],
    )


def _dot(a, w):
    return jnp.dot(a, w, preferred_element_type=jnp.float32)


def _mlp3(t, w1, b1, w2, b2, w3, b3):
    t = jnp.maximum(_dot(t, w1) + b1, 0.0)
    t = jnp.maximum(_dot(t, w2) + b2, 0.0)
    return _dot(t, w3) + b3


def _node_embed_body(x_ref, w1, b1, w2, b2, w3, b3, o_ref):
    o_ref[...] = _mlp3(x_ref[...], w1[...], b1[...], w2[...], b2[...],
                       w3[...], b3[...])


def _edge_embed_body(xi_ref, xj_ref, wa, wb, wr, b1, w2, b2, w3, b3, o_ref):
    xi = xi_ref[...]
    xj = xj_ref[...]
    dx = xi - xj
    m = (lax.broadcasted_iota(jnp.int32, (1, 16), 1) < 2).astype(jnp.float32)
    s = jnp.sum(dx * dx * m, axis=1, keepdims=True)
    r = jnp.sqrt(s) * (1.0 / RADIUS)
    t = jnp.maximum(_dot(xi, wa[...]) + _dot(xj, wb[...]) + r * wr[...]
                    + b1[...], 0.0)
    t = jnp.maximum(_dot(t, w2[...]) + b2[...], 0.0)
    o_ref[...] = _dot(t, w3[...]) + b3[...]


def _edge_layer_body(e_ref, hd_ref, hs_ref, w1a, w1b, w1c, b1, w2, b2, w3, b3,
                     msg_ref, enew_ref):
    i = pl.program_id(0)
    eb = e_ref[...]
    t = jnp.maximum(_dot(eb, w1a[...]) + _dot(hd_ref[...], w1b[...])
                    + _dot(hs_ref[...], w1c[...]) + b1[...], 0.0)
    t = jnp.maximum(_dot(t, w2[...]) + b2[...], 0.0)
    msg = _dot(t, w3[...]) + b3[...]
    msg_ref[...] = msg
    enew_ref[...] = eb + msg

    @pl.when(i >= E // BE)
    def _():
        row = i * BE + lax.broadcasted_iota(jnp.int32, (BE, 1), 0)
        msg_ref[...] = jnp.where(row < E, msg, 0.0)


def _node_layer_body(h_ref, p0_ref, p1_ref, w1a, w1b, b1, w2, b2, w3, b3,
                     o_ref):
    h = h_ref[...]
    aggr = p0_ref[0] + p1_ref[0]
    t = jnp.maximum(_dot(h, w1a[...]) + _dot(aggr, w1b[...]) + b1[...], 0.0)
    t = jnp.maximum(_dot(t, w2[...]) + b2[...], 0.0)
    o_ref[...] = h + _dot(t, w3[...]) + b3[...]


def _node_out_body(h_ref, w1, b1, w2, b2, w3, b3, o_ref):
    o_ref[...] = _mlp3(h_ref[...], w1[...], b1[...], w2[...], b2[...],
                       w3[...], b3[...])


def _wspecs(ws):
    return [pl.BlockSpec(w.shape, lambda i, nd=w.ndim: (0,) * nd) for w in ws]


def _edge_call(body, ws, n_out):
    eb = pl.BlockSpec((BE, EMB), lambda i: (i, 0))
    out_shape = [jax.ShapeDtypeStruct((EPAD, EMB), jnp.float32)] * n_out
    return pl.pallas_call(
        body,
        grid=(NEB,),
        in_specs=[eb,
                  pl.BlockSpec((BE, EMB), lambda i: (i, 0)),
                  pl.BlockSpec((BE, EMB), lambda i: (i + NEB, 0))] + _wspecs(ws),
        out_specs=[eb] * n_out,
        out_shape=out_shape,
    )


def kernel(x, edge_index, edge_attr, params):
    del edge_attr
    src = edge_index[0]
    dst = edge_index[1]
    zeros_i = jnp.zeros((EPAD - E,), jnp.int32)
    dst_p = jnp.concatenate([dst, zeros_i])
    src_p = jnp.concatenate([src, zeros_i])
    gidx = jnp.concatenate([dst_p, src_p]).reshape(M2 // GCH, GCH)
    dst2d = dst_p.reshape(EPAD // GCH, GCH)
    xpad = jnp.pad(x, ((0, 0), (0, 11)))
    zrows = jnp.zeros((_ZCH, EMB), jnp.float32)

    (wn1, bn1), (wn2, bn2), (wn3, bn3) = params['embedding_node']
    cn = jnp.concatenate([wn1[0:4], jnp.sum(wn1[4:8], axis=0, keepdims=True)])
    p_ne = [cn, bn1, wn2, bn2, wn3, bn3]

    (we1, be1), (we2, be2), (we3, be3) = params['embedding_edges']
    z = jnp.zeros((1, EMB), jnp.float32)
    inv_r = 1.0 / RADIUS
    wa = jnp.concatenate(
        [we1[0:2] * inv_r, we1[3:5],
         jnp.sum(we1[7:11], axis=0, keepdims=True)] + [z] * 11)
    wb = jnp.concatenate([-we1[0:2] * inv_r, we1[5:7], z] + [z] * 11)
    wr = we1[2:3]
    p_ee = [wa, wb, wr, be1, we2, be2, we3, be3]
    p_out = [t for wb_ in params['node_out'] for t in wb_]

    gx = _make_gather(N, 16, M2)(xpad, gidx)

    h = pl.pallas_call(
        _node_embed_body,
        grid=(NNB,),
        in_specs=[pl.BlockSpec((BN, 5), lambda i: (i, 0))] + _wspecs(p_ne),
        out_specs=pl.BlockSpec((BN, EMB), lambda i: (i, 0)),
        out_shape=jax.ShapeDtypeStruct((N, EMB), jnp.float32),
    )(x, *p_ne)

    e = pl.pallas_call(
        _edge_embed_body,
        grid=(NEB,),
        in_specs=[pl.BlockSpec((BE, 16), lambda i: (i, 0)),
                  pl.BlockSpec((BE, 16), lambda i: (i + NEB, 0))] + _wspecs(p_ee),
        out_specs=pl.BlockSpec((BE, EMB), lambda i: (i, 0)),
        out_shape=jax.ShapeDtypeStruct((EPAD, EMB), jnp.float32),
    )(gx, gx, *p_ee)

    gather_h = _make_gather(N, EMB, M2)
    scatter = _make_scatter()

    for lp in params['layers']:
        (le1, lb1), (le2, lb2), (le3, lb3) = lp['lin_edge']
        p_edge = [le1[0:EMB], le1[EMB:2 * EMB], le1[2 * EMB:], lb1,
                  le2, lb2, le3, lb3]
        (ln1, nb1), (ln2, nb2), (ln3, nb3) = lp['lin_node']
        p_node = [ln1[0:EMB], ln1[EMB:], nb1, ln2, nb2, ln3, nb3]

        gh = gather_h(h, gidx)
        msg, e = _edge_call(_edge_layer_body, p_edge, 2)(e, gh, gh, *p_edge)
        partials = scatter(msg, dst2d, zrows)
        h = pl.pallas_call(
            _node_layer_body,
            grid=(NNB,),
            in_specs=[pl.BlockSpec((BN, EMB), lambda i: (i, 0)),
                      pl.BlockSpec((1, BN, EMB), lambda i: (0, i, 0)),
                      pl.BlockSpec((1, BN, EMB), lambda i: (1, i, 0))] +
                     _wspecs(p_node),
            out_specs=pl.BlockSpec((BN, EMB), lambda i: (i, 0)),
            out_shape=jax.ShapeDtypeStruct((N, EMB), jnp.float32),
        )(h, partials, partials, *p_node)

    pred = pl.pallas_call(
        _node_out_body,
        grid=(NNB,),
        in_specs=[pl.BlockSpec((BN, EMB), lambda i: (i, 0))] + _wspecs(p_out),
        out_specs=pl.BlockSpec((BN, 2), lambda i: (i, 0)),
        out_shape=jax.ShapeDtypeStruct((N, 2), jnp.float32),
    )(h, *p_out)
    return pred


# speedup vs baseline: 2.4394x; 1.0708x over previous
import functools

import jax
import jax.numpy as jnp
from jax import lax
from jax.experimental import pallas as pl
from jax.experimental.pallas import tpu as pltpu
from jax.experimental.pallas import tpu_sc as plsc

_SC_PARAMS = pltpu.CompilerParams(use_tc_tiling_on_sc=False)

N = 50000
E = 800000
EMB = 32
RADIUS = 0.075

NC = 2
NS = 16
NW = NC * NS
GCH = 128
K2 = 8

EPAD = 819200
M2 = 2 * EPAD
NPAD = 50176
BE = 8192
BN = 2048
NEB = EPAD // BE
NNB = (N + BN - 1) // BN


def _gather_body(rounds, table_hbm, idx_hbm, out_hbm, idx_v, rows_v,
                 sg0, sg1, so0, so1):
    wid = lax.axis_index("c") * NS + lax.axis_index("s")
    base = wid * (rounds * K2)
    pltpu.sync_copy(idx_hbm.at[pl.ds(base, rounds * K2)], idx_v)
    sg = (sg0, sg1)
    so = (so0, so1)

    def fire(r, t):
        for k in range(K2):
            pltpu.async_copy(
                table_hbm.at[idx_v.at[r * K2 + k]],
                rows_v.at[t, pl.ds(k * GCH, GCH)], sg[t])

    def drain_gathers(t):
        for _ in range(K2):
            pltpu.make_async_copy(
                table_hbm.at[idx_v.at[0]], rows_v.at[t, pl.ds(0, GCH)],
                sg[t]).wait()

    fire(0, 0)

    @pl.loop(0, rounds, step=2)
    def _(r0):
        for t in range(2):
            r = r0 + t
            nxt = 1 - t

            @pl.when(r + 1 < rounds)
            def _():
                @pl.when(r >= 1)
                def _():
                    pltpu.make_async_copy(
                        rows_v.at[nxt], out_hbm.at[pl.ds(0, K2 * GCH)],
                        so[nxt]).wait()

                fire(r + 1, nxt)

            drain_gathers(t)
            pltpu.async_copy(
                rows_v.at[t],
                out_hbm.at[pl.ds((base + r * K2) * GCH, K2 * GCH)], so[t])

    for t in range(2):
        pltpu.make_async_copy(
            rows_v.at[t], out_hbm.at[pl.ds(0, K2 * GCH)], so[t]).wait()


def _make_gather(n_rows, d, m):
    rounds = m // (NW * GCH * K2)
    mesh = plsc.VectorSubcoreMesh(
        core_axis_name="c", subcore_axis_name="s", num_cores=NC, num_subcores=NS)
    return pl.kernel(
        functools.partial(_gather_body, rounds),
        out_type=jax.ShapeDtypeStruct((m, d), jnp.float32),
        mesh=mesh,
        compiler_params=_SC_PARAMS,
        scratch_types=[
            pltpu.VMEM((m // (NW * GCH), GCH), jnp.int32),
            pltpu.VMEM((2, K2 * GCH, d), jnp.float32),
            pltpu.SemaphoreType.DMA,
            pltpu.SemaphoreType.DMA,
            pltpu.SemaphoreType.DMA,
            pltpu.SemaphoreType.DMA,
        ],
    )


_TPW = NPAD // NS
_ZCH = _TPW // 7
_SG = 4


def _scatter_body(msg_hbm, idx_hbm, zeros_hbm, out_hbm, idx_v, rows_v, acc):
    c = lax.axis_index("c")
    s = lax.axis_index("s")
    pltpu.sync_copy(zeros_hbm, rows_v.at[pl.ds(0, _ZCH)])
    for q in range(7):
        pltpu.sync_copy(rows_v.at[pl.ds(0, _ZCH)],
                        acc.at[pl.ds(s * _TPW + q * _ZCH, _ZCH)])
    plsc.subcore_barrier()

    chunks = EPAD // (NW * GCH)
    base = (c * NS + s) * chunks

    @pl.loop(0, chunks // _SG)
    def _(g):
        pltpu.sync_copy(idx_hbm.at[pl.ds(base + g * _SG, _SG)], idx_v)
        pltpu.sync_copy(
            msg_hbm.at[pl.ds((base + g * _SG) * GCH, _SG * GCH)], rows_v)
        for k in range(_SG):
            pltpu.sync_copy(
                rows_v.at[pl.ds(k * GCH, GCH)],
                acc.at[idx_v.at[k]], add=True)

    plsc.subcore_barrier()
    for q in range(7):
        row0 = s * _TPW + q * _ZCH
        pltpu.sync_copy(acc.at[pl.ds(row0, _ZCH)], rows_v.at[pl.ds(0, _ZCH)])
        pltpu.sync_copy(rows_v.at[pl.ds(0, _ZCH)],
                        out_hbm.at[c].at[pl.ds(row0, _ZCH)])


def _make_scatter():
    mesh = plsc.VectorSubcoreMesh(
        core_axis_name="c", subcore_axis_name="s", num_cores=NC, num_subcores=NS)
    return pl.kernel(
        _scatter_body,
        out_type=jax.ShapeDtypeStruct((NC, NPAD, EMB), jnp.float32),
        mesh=mesh,
        compiler_params=_SC_PARAMS,
        scratch_types=[
            pltpu.VMEM((_SG, GCH), jnp.int32),
            pltpu.VMEM((_SG * GCH, EMB), jnp.float32),
            pltpu.VMEM_SHARED((NPAD, EMB), jnp.float32),
        ],
    )


def _dot(a, w):
    return jnp.dot(a, w, preferred_element_type=jnp.float32)


def _mlp3(t, w1, b1, w2, b2, w3, b3):
    t = jnp.maximum(_dot(t, w1) + b1, 0.0)
    t = jnp.maximum(_dot(t, w2) + b2, 0.0)
    return _dot(t, w3) + b3


def _node_embed_body(x_ref, w1, b1, w2, b2, w3, b3, o_ref):
    o_ref[...] = _mlp3(x_ref[...], w1[...], b1[...], w2[...], b2[...],
                       w3[...], b3[...])


def _edge_embed_body(xi_ref, xj_ref, wa, wb, wr, b1, w2, b2, w3, b3, o_ref):
    xi = xi_ref[...]
    xj = xj_ref[...]
    dx = xi - xj
    m = (lax.broadcasted_iota(jnp.int32, (1, 16), 1) < 2).astype(jnp.float32)
    s = jnp.sum(dx * dx * m, axis=1, keepdims=True)
    r = jnp.sqrt(s) * (1.0 / RADIUS)
    t = jnp.maximum(_dot(xi, wa[...]) + _dot(xj, wb[...]) + r * wr[...]
                    + b1[...], 0.0)
    t = jnp.maximum(_dot(t, w2[...]) + b2[...], 0.0)
    o_ref[...] = _dot(t, w3[...]) + b3[...]


def _edge_layer_body(e_ref, hd_ref, hs_ref, w1a, w1b, w1c, b1, w2, b2, w3, b3,
                     msg_ref, enew_ref):
    i = pl.program_id(0)
    eb = e_ref[...]
    t = jnp.maximum(_dot(eb, w1a[...]) + _dot(hd_ref[...], w1b[...])
                    + _dot(hs_ref[...], w1c[...]) + b1[...], 0.0)
    t = jnp.maximum(_dot(t, w2[...]) + b2[...], 0.0)
    msg = _dot(t, w3[...]) + b3[...]
    msg_ref[...] = msg
    enew_ref[...] = eb + msg

    @pl.when(i >= E // BE)
    def _():
        row = i * BE + lax.broadcasted_iota(jnp.int32, (BE, 1), 0)
        msg_ref[...] = jnp.where(row < E, msg, 0.0)


def _node_layer_body(h_ref, p0_ref, p1_ref, w1a, w1b, b1, w2, b2, w3, b3,
                     o_ref):
    h = h_ref[...]
    aggr = p0_ref[0] + p1_ref[0]
    t = jnp.maximum(_dot(h, w1a[...]) + _dot(aggr, w1b[...]) + b1[...], 0.0)
    t = jnp.maximum(_dot(t, w2[...]) + b2[...], 0.0)
    o_ref[...] = h + _dot(t, w3[...]) + b3[...]


def _node_out_body(h_ref, w1, b1, w2, b2, w3, b3, o_ref):
    o_ref[...] = _mlp3(h_ref[...], w1[...], b1[...], w2[...], b2[...],
                       w3[...], b3[...])


def _wspecs(ws):
    return [pl.BlockSpec(w.shape, lambda i, nd=w.ndim: (0,) * nd) for w in ws]


def _edge_call(body, ws, n_out):
    eb = pl.BlockSpec((BE, EMB), lambda i: (i, 0))
    out_shape = [jax.ShapeDtypeStruct((EPAD, EMB), jnp.float32)] * n_out
    return pl.pallas_call(
        body,
        grid=(NEB,),
        in_specs=[eb,
                  pl.BlockSpec((BE, EMB), lambda i: (i, 0)),
                  pl.BlockSpec((BE, EMB), lambda i: (i + NEB, 0))] + _wspecs(ws),
        out_specs=[eb] * n_out,
        out_shape=out_shape,
    )


def kernel(x, edge_index, edge_attr, params):
    del edge_attr
    src = edge_index[0]
    dst = edge_index[1]
    zeros_i = jnp.zeros((EPAD - E,), jnp.int32)
    dst_p = jnp.concatenate([dst, zeros_i])
    src_p = jnp.concatenate([src, zeros_i])
    gidx = jnp.concatenate([dst_p, src_p]).reshape(M2 // GCH, GCH)
    dst2d = dst_p.reshape(EPAD // GCH, GCH)
    xpad = jnp.pad(x, ((0, 0), (0, 11)))
    zrows = jnp.zeros((_ZCH, EMB), jnp.float32)

    (wn1, bn1), (wn2, bn2), (wn3, bn3) = params['embedding_node']
    cn = jnp.concatenate([wn1[0:4], jnp.sum(wn1[4:8], axis=0, keepdims=True)])
    p_ne = [cn, bn1, wn2, bn2, wn3, bn3]

    (we1, be1), (we2, be2), (we3, be3) = params['embedding_edges']
    z = jnp.zeros((1, EMB), jnp.float32)
    inv_r = 1.0 / RADIUS
    wa = jnp.concatenate(
        [we1[0:2] * inv_r, we1[3:5],
         jnp.sum(we1[7:11], axis=0, keepdims=True)] + [z] * 11)
    wb = jnp.concatenate([-we1[0:2] * inv_r, we1[5:7], z] + [z] * 11)
    wr = we1[2:3]
    p_ee = [wa, wb, wr, be1, we2, be2, we3, be3]
    p_out = [t for wb_ in params['node_out'] for t in wb_]

    gx = _make_gather(N, 16, M2)(xpad, gidx)

    h = pl.pallas_call(
        _node_embed_body,
        grid=(NNB,),
        in_specs=[pl.BlockSpec((BN, 5), lambda i: (i, 0))] + _wspecs(p_ne),
        out_specs=pl.BlockSpec((BN, EMB), lambda i: (i, 0)),
        out_shape=jax.ShapeDtypeStruct((N, EMB), jnp.float32),
    )(x, *p_ne)

    e = pl.pallas_call(
        _edge_embed_body,
        grid=(NEB,),
        in_specs=[pl.BlockSpec((BE, 16), lambda i: (i, 0)),
                  pl.BlockSpec((BE, 16), lambda i: (i + NEB, 0))] + _wspecs(p_ee),
        out_specs=pl.BlockSpec((BE, EMB), lambda i: (i, 0)),
        out_shape=jax.ShapeDtypeStruct((EPAD, EMB), jnp.float32),
    )(gx, gx, *p_ee)

    gather_h = _make_gather(N, EMB, M2)
    scatter = _make_scatter()

    for lp in params['layers']:
        (le1, lb1), (le2, lb2), (le3, lb3) = lp['lin_edge']
        p_edge = [le1[0:EMB], le1[EMB:2 * EMB], le1[2 * EMB:], lb1,
                  le2, lb2, le3, lb3]
        (ln1, nb1), (ln2, nb2), (ln3, nb3) = lp['lin_node']
        p_node = [ln1[0:EMB], ln1[EMB:], nb1, ln2, nb2, ln3, nb3]

        gh = gather_h(h, gidx)
        msg, e = _edge_call(_edge_layer_body, p_edge, 2)(e, gh, gh, *p_edge)
        partials = scatter(msg, dst2d, zrows)
        h = pl.pallas_call(
            _node_layer_body,
            grid=(NNB,),
            in_specs=[pl.BlockSpec((BN, EMB), lambda i: (i, 0)),
                      pl.BlockSpec((1, BN, EMB), lambda i: (0, i, 0)),
                      pl.BlockSpec((1, BN, EMB), lambda i: (1, i, 0))] +
                     _wspecs(p_node),
            out_specs=pl.BlockSpec((BN, EMB), lambda i: (i, 0)),
            out_shape=jax.ShapeDtypeStruct((N, EMB), jnp.float32),
        )(h, partials, partials, *p_node)

    pred = pl.pallas_call(
        _node_out_body,
        grid=(NNB,),
        in_specs=[pl.BlockSpec((BN, EMB), lambda i: (i, 0))] + _wspecs(p_out),
        out_specs=pl.BlockSpec((BN, 2), lambda i: (i, 0)),
        out_shape=jax.ShapeDtypeStruct((N, 2), jnp.float32),
    )(h, *p_out)
    return pred


# speedup vs baseline: 2.6381x; 1.0815x over previous
import functools

import jax
import jax.numpy as jnp
from jax import lax
from jax.experimental import pallas as pl
from jax.experimental.pallas import tpu as pltpu
from jax.experimental.pallas import tpu_sc as plsc

_SC_PARAMS = pltpu.CompilerParams(use_tc_tiling_on_sc=False)

N = 50000
E = 800000
EMB = 32
RADIUS = 0.075

NC = 2
NS = 16
NW = NC * NS
GCH = 128
K2 = 8

EPAD = 819200
M2 = 2 * EPAD
NPAD = 50176
BE = 8192
BN = 2048
NEB = EPAD // BE
NNB = (N + BN - 1) // BN


def _gather_body(rounds, table_hbm, idx_hbm, out_hbm, idx_v, rows_v,
                 sg0, sg1, so0, so1):
    wid = lax.axis_index("c") * NS + lax.axis_index("s")
    base = wid * (rounds * K2)
    pltpu.sync_copy(idx_hbm.at[pl.ds(base, rounds * K2)], idx_v)
    sg = (sg0, sg1)
    so = (so0, so1)

    def fire(r, t):
        for k in range(K2):
            pltpu.async_copy(
                table_hbm.at[idx_v.at[r * K2 + k]],
                rows_v.at[t, pl.ds(k * GCH, GCH)], sg[t])

    def drain_gathers(t):
        for _ in range(K2):
            pltpu.make_async_copy(
                table_hbm.at[idx_v.at[0]], rows_v.at[t, pl.ds(0, GCH)],
                sg[t]).wait()

    fire(0, 0)

    @pl.loop(0, rounds, step=2)
    def _(r0):
        for t in range(2):
            r = r0 + t
            nxt = 1 - t

            @pl.when(r + 1 < rounds)
            def _():
                @pl.when(r >= 1)
                def _():
                    pltpu.make_async_copy(
                        rows_v.at[nxt], out_hbm.at[pl.ds(0, K2 * GCH)],
                        so[nxt]).wait()

                fire(r + 1, nxt)

            drain_gathers(t)
            pltpu.async_copy(
                rows_v.at[t],
                out_hbm.at[pl.ds((base + r * K2) * GCH, K2 * GCH)], so[t])

    for t in range(2):
        pltpu.make_async_copy(
            rows_v.at[t], out_hbm.at[pl.ds(0, K2 * GCH)], so[t]).wait()


def _make_gather(n_rows, d, m, dtype=jnp.float32):
    rounds = m // (NW * GCH * K2)
    mesh = plsc.VectorSubcoreMesh(
        core_axis_name="c", subcore_axis_name="s", num_cores=NC, num_subcores=NS)
    return pl.kernel(
        functools.partial(_gather_body, rounds),
        out_type=jax.ShapeDtypeStruct((m, d), dtype),
        mesh=mesh,
        compiler_params=_SC_PARAMS,
        scratch_types=[
            pltpu.VMEM((m // (NW * GCH), GCH), jnp.int32),
            pltpu.VMEM((2, K2 * GCH, d), dtype),
            pltpu.SemaphoreType.DMA,
            pltpu.SemaphoreType.DMA,
            pltpu.SemaphoreType.DMA,
            pltpu.SemaphoreType.DMA,
        ],
    )


_TPW = NPAD // NS
_ZCH = _TPW // 7
_SG = 4


def _scatter_body(msg_hbm, idx_hbm, zeros_hbm, out_hbm, idx_v, rows_v, acc):
    c = lax.axis_index("c")
    s = lax.axis_index("s")
    pltpu.sync_copy(zeros_hbm, rows_v.at[pl.ds(0, _ZCH)])
    for q in range(7):
        pltpu.sync_copy(rows_v.at[pl.ds(0, _ZCH)],
                        acc.at[pl.ds(s * _TPW + q * _ZCH, _ZCH)])
    plsc.subcore_barrier()

    chunks = EPAD // (NW * GCH)
    base = (c * NS + s) * chunks

    @pl.loop(0, chunks // _SG)
    def _(g):
        pltpu.sync_copy(idx_hbm.at[pl.ds(base + g * _SG, _SG)], idx_v)
        pltpu.sync_copy(
            msg_hbm.at[pl.ds((base + g * _SG) * GCH, _SG * GCH)], rows_v)
        for k in range(_SG):
            pltpu.sync_copy(
                rows_v.at[pl.ds(k * GCH, GCH)],
                acc.at[idx_v.at[k]], add=True)

    plsc.subcore_barrier()
    for q in range(7):
        row0 = s * _TPW + q * _ZCH
        pltpu.sync_copy(acc.at[pl.ds(row0, _ZCH)], rows_v.at[pl.ds(0, _ZCH)])
        pltpu.sync_copy(rows_v.at[pl.ds(0, _ZCH)],
                        out_hbm.at[c].at[pl.ds(row0, _ZCH)])


def _make_scatter():
    mesh = plsc.VectorSubcoreMesh(
        core_axis_name="c", subcore_axis_name="s", num_cores=NC, num_subcores=NS)
    return pl.kernel(
        _scatter_body,
        out_type=jax.ShapeDtypeStruct((NC, NPAD, EMB), jnp.float32),
        mesh=mesh,
        compiler_params=_SC_PARAMS,
        scratch_types=[
            pltpu.VMEM((_SG, GCH), jnp.int32),
            pltpu.VMEM((_SG * GCH, EMB), jnp.float32),
            pltpu.VMEM_SHARED((NPAD, EMB), jnp.float32),
        ],
    )


def _dot(a, w):
    return jnp.dot(a.astype(jnp.bfloat16), w, preferred_element_type=jnp.float32)


def _mlp3(t, w1, b1, w2, b2, w3, b3):
    t = jnp.maximum(_dot(t, w1) + b1, 0.0)
    t = jnp.maximum(_dot(t, w2) + b2, 0.0)
    return _dot(t, w3) + b3


def _node_embed_body(x_ref, w1, b1, w2, b2, w3, b3, o_ref, ob_ref):
    x = x_ref[...]
    nf = jnp.concatenate([x[:, 0:4]] + [x[:, 4:5]] * 4, axis=-1)
    h = _mlp3(nf, w1[...], b1[...], w2[...], b2[...], w3[...], b3[...])
    o_ref[...] = h
    ob_ref[...] = h.astype(jnp.bfloat16)


def _edge_embed_body(xi_ref, xj_ref, w1, b1, w2, b2, w3, b3, o_ref):
    xi = xi_ref[...]
    xj = xj_ref[...]
    dpos = xi[:, 0:2] - xj[:, 0:2]
    r = jnp.sqrt(jnp.sum(dpos * dpos, axis=1, keepdims=True)) / RADIUS
    feat = jnp.concatenate(
        [dpos / RADIUS, r, xi[:, 2:4], xj[:, 2:4]] + [xi[:, 4:5]] * 4 +
        [jnp.zeros_like(xi[:, 0:5])], axis=-1)
    o_ref[...] = _mlp3(feat, w1[...], b1[...], w2[...], b2[...], w3[...],
                       b3[...])


def _edge_layer_body(e_ref, hd_ref, hs_ref, w1, b1, w2, b2, w3, b3,
                     msg_ref, enew_ref):
    i = pl.program_id(0)
    eb = e_ref[...]
    t = jnp.concatenate(
        [eb.astype(jnp.bfloat16), hd_ref[...], hs_ref[...]], axis=-1)
    msg = _mlp3(t, w1[...], b1[...], w2[...], b2[...], w3[...], b3[...])
    msg_ref[...] = msg
    enew_ref[...] = eb + msg

    @pl.when(i >= E // BE)
    def _():
        row = i * BE + lax.broadcasted_iota(jnp.int32, (BE, 1), 0)
        msg_ref[...] = jnp.where(row < E, msg, 0.0)


def _node_layer_body(h_ref, p0_ref, p1_ref, w1, b1, w2, b2, w3, b3,
                     o_ref, ob_ref):
    h = h_ref[...]
    aggr = p0_ref[0] + p1_ref[0]
    t = jnp.concatenate(
        [h.astype(jnp.bfloat16), aggr.astype(jnp.bfloat16)], axis=-1)
    hn = h + _mlp3(t, w1[...], b1[...], w2[...], b2[...], w3[...], b3[...])
    o_ref[...] = hn
    ob_ref[...] = hn.astype(jnp.bfloat16)


def _node_out_body(h_ref, w1, b1, w2, b2, w3, b3, o_ref):
    o_ref[...] = _mlp3(h_ref[...], w1[...], b1[...], w2[...], b2[...],
                       w3[...], b3[...])


def _wspecs(ws):
    return [pl.BlockSpec(w.shape, lambda i, nd=w.ndim: (0,) * nd) for w in ws]


def _edge_call(body, ws, n_out):
    eb = pl.BlockSpec((BE, EMB), lambda i: (i, 0))
    out_shape = [jax.ShapeDtypeStruct((EPAD, EMB), jnp.float32)] * n_out
    return pl.pallas_call(
        body,
        grid=(NEB,),
        in_specs=[eb,
                  pl.BlockSpec((BE, EMB), lambda i: (i, 0)),
                  pl.BlockSpec((BE, EMB), lambda i: (i + NEB, 0))] + _wspecs(ws),
        out_specs=[eb] * n_out,
        out_shape=out_shape,
    )


def kernel(x, edge_index, edge_attr, params):
    del edge_attr
    src = edge_index[0]
    dst = edge_index[1]
    zeros_i = jnp.zeros((EPAD - E,), jnp.int32)
    dst_p = jnp.concatenate([dst, zeros_i])
    src_p = jnp.concatenate([src, zeros_i])
    gidx = jnp.concatenate([dst_p, src_p]).reshape(M2 // GCH, GCH)
    dst2d = dst_p.reshape(EPAD // GCH, GCH)
    xpad = jnp.pad(x, ((0, 0), (0, 11)))
    zrows = jnp.zeros((_ZCH, EMB), jnp.float32)

    bf16 = jnp.bfloat16

    def wb16(w):
        return w.astype(bf16)

    (wn1, bn1), (wn2, bn2), (wn3, bn3) = params['embedding_node']
    p_ne = [wb16(wn1), bn1, wb16(wn2), bn2, wb16(wn3), bn3]

    (we1, be1), (we2, be2), (we3, be3) = params['embedding_edges']
    we1p = jnp.pad(we1, ((0, 5), (0, 0)))
    p_ee = [wb16(we1p), be1, wb16(we2), be2, wb16(we3), be3]

    (wo1, bo1), (wo2, bo2), (wo3, bo3) = params['node_out']
    p_out = [wb16(wo1), bo1, wb16(wo2), bo2, wb16(wo3), bo3]

    gx = _make_gather(N, 16, M2)(xpad, gidx)

    nb_spec = pl.BlockSpec((BN, EMB), lambda i: (i, 0))
    h, hb = pl.pallas_call(
        _node_embed_body,
        grid=(NNB,),
        in_specs=[pl.BlockSpec((BN, 5), lambda i: (i, 0))] + _wspecs(p_ne),
        out_specs=[nb_spec, nb_spec],
        out_shape=[jax.ShapeDtypeStruct((N, EMB), jnp.float32),
                   jax.ShapeDtypeStruct((N, EMB), jnp.bfloat16)],
    )(x, *p_ne)

    e = pl.pallas_call(
        _edge_embed_body,
        grid=(NEB,),
        in_specs=[pl.BlockSpec((BE, 16), lambda i: (i, 0)),
                  pl.BlockSpec((BE, 16), lambda i: (i + NEB, 0))] + _wspecs(p_ee),
        out_specs=pl.BlockSpec((BE, EMB), lambda i: (i, 0)),
        out_shape=jax.ShapeDtypeStruct((EPAD, EMB), jnp.float32),
    )(gx, gx, *p_ee)

    gather_h = _make_gather(N, EMB, M2, jnp.bfloat16)
    scatter = _make_scatter()

    for lp in params['layers']:
        (le1, lb1), (le2, lb2), (le3, lb3) = lp['lin_edge']
        p_edge = [wb16(le1), lb1, wb16(le2), lb2, wb16(le3), lb3]
        (ln1, nb1), (ln2, nb2), (ln3, nb3) = lp['lin_node']
        p_node = [wb16(ln1), nb1, wb16(ln2), nb2, wb16(ln3), nb3]

        gh = gather_h(hb, gidx)
        msg, e = _edge_call(_edge_layer_body, p_edge, 2)(e, gh, gh, *p_edge)
        partials = scatter(msg, dst2d, zrows)
        h, hb = pl.pallas_call(
            _node_layer_body,
            grid=(NNB,),
            in_specs=[pl.BlockSpec((BN, EMB), lambda i: (i, 0)),
                      pl.BlockSpec((1, BN, EMB), lambda i: (0, i, 0)),
                      pl.BlockSpec((1, BN, EMB), lambda i: (1, i, 0))] +
                     _wspecs(p_node),
            out_specs=[nb_spec, nb_spec],
            out_shape=[jax.ShapeDtypeStruct((N, EMB), jnp.float32),
                       jax.ShapeDtypeStruct((N, EMB), jnp.bfloat16)],
        )(h, partials, partials, *p_node)

    pred = pl.pallas_call(
        _node_out_body,
        grid=(NNB,),
        in_specs=[pl.BlockSpec((BN, EMB), lambda i: (i, 0))] + _wspecs(p_out),
        out_specs=pl.BlockSpec((BN, 2), lambda i: (i, 0)),
        out_shape=jax.ShapeDtypeStruct((N, 2), jnp.float32),
    )(h, *p_out)
    return pred
